# baseline (device time: 32836 ns/iter reference)
import jax
import jax.numpy as jnp
from jax import lax
from jax.experimental import pallas as pl
from jax.experimental.pallas import tpu as pltpu

N_DEV = 4
V_PER = 4096
T = 512
D = 512


def kernel(ids, E):
    ids2 = jnp.reshape(ids, (T, 1))

    def body(ids_ref, e_ref, out_ref, comm_ref, send_sems, recv_sems):
        my_pos = lax.axis_index("i")
        left = (my_pos - 1) % N_DEV
        right = (my_pos + 1) % N_DEV

        barrier_sem = pltpu.get_barrier_semaphore()
        for nbr in [left, right]:
            pl.semaphore_signal(
                barrier_sem, inc=1,
                device_id=(nbr,), device_id_type=pl.DeviceIdType.MESH,
            )
        pl.semaphore_wait(barrier_sem, 2)

        local_id = ids_ref[:, :] - my_pos * V_PER
        iota = lax.broadcasted_iota(jnp.int32, (T, V_PER), 1)
        onehot = (iota == local_id).astype(jnp.bfloat16)
        e_bf16 = e_ref[:, :].astype(jnp.bfloat16)
        partial = jnp.dot(onehot, e_bf16, preferred_element_type=jnp.float32)

        acc = partial
        comm_ref[0, :, :] = partial.astype(jnp.bfloat16)

        for h in range(N_DEV - 1):
            send_slot = h % 2
            recv_slot = (h + 1) % 2
            rdma = pltpu.make_async_remote_copy(
                src_ref=comm_ref.at[send_slot],
                dst_ref=comm_ref.at[recv_slot],
                send_sem=send_sems.at[send_slot],
                recv_sem=recv_sems.at[recv_slot],
                device_id=(right,),
                device_id_type=pl.DeviceIdType.MESH,
            )
            rdma.start()
            rdma.wait()
            acc = acc + comm_ref[recv_slot, :, :].astype(jnp.float32)

        out_ref[:, :] = acc

    return pl.pallas_call(
        body,
        out_shape=jax.ShapeDtypeStruct((T, D), jnp.float32),
        in_specs=[
            pl.BlockSpec(memory_space=pltpu.VMEM),
            pl.BlockSpec(memory_space=pltpu.VMEM),
        ],
        out_specs=pl.BlockSpec(memory_space=pltpu.VMEM),
        scratch_shapes=[
            pltpu.VMEM((2, T, D), jnp.bfloat16),
            pltpu.SemaphoreType.DMA((2,)),
            pltpu.SemaphoreType.DMA((2,)),
        ],
        compiler_params=pltpu.CompilerParams(collective_id=0),
    )(ids2, E)


# device time: 19039 ns/iter; 1.7247x vs baseline; 1.7247x over previous
import jax
import jax.numpy as jnp
from jax import lax
from jax.experimental import pallas as pl
from jax.experimental.pallas import tpu as pltpu

N_DEV = 4
V_PER = 4096
T = 512
TB = T // N_DEV
D = 512


def kernel(ids, E):
    ids2 = jnp.reshape(ids, (T, 1))

    def body(ids_ref, e_ref, out_ref,
             rs_send_buf, rs_buf, ag_send_buf, ag_buf,
             rs_send_sems, rs_recv_sems, ag_send_sems, ag_recv_sems):
        my_pos = lax.axis_index("i")

        barrier_sem = pltpu.get_barrier_semaphore()
        for d in range(1, N_DEV):
            pl.semaphore_signal(
                barrier_sem, inc=1,
                device_id=((my_pos + d) % N_DEV,),
                device_id_type=pl.DeviceIdType.MESH,
            )
        pl.semaphore_wait(barrier_sem, N_DEV - 1)

        e_bf16 = e_ref[:, :].astype(jnp.bfloat16)
        base = my_pos * V_PER

        def partial_block(blk):
            loc = ids_ref[pl.ds(blk * TB, TB), :] - base
            iota = lax.broadcasted_iota(jnp.int32, (TB, V_PER), 1)
            onehot = (iota == loc).astype(jnp.bfloat16)
            return jnp.dot(onehot, e_bf16,
                           preferred_element_type=jnp.float32)

        rs_rdmas = []
        for d in range(1, N_DEV):
            peer = (my_pos + d) % N_DEV
            rs_send_buf[d - 1, :, :] = partial_block(peer).astype(jnp.bfloat16)
            rdma = pltpu.make_async_remote_copy(
                src_ref=rs_send_buf.at[d - 1],
                dst_ref=rs_buf.at[d - 1],
                send_sem=rs_send_sems.at[d - 1],
                recv_sem=rs_recv_sems.at[d - 1],
                device_id=(peer,),
                device_id_type=pl.DeviceIdType.MESH,
            )
            rdma.start()
            rs_rdmas.append(rdma)

        acc = partial_block(my_pos)

        for d in range(1, N_DEV):
            rs_rdmas[d - 1].wait_recv()
            acc = acc + rs_buf[d - 1, :, :].astype(jnp.float32)

        ag_send_buf[:, :] = acc.astype(jnp.bfloat16)
        ag_rdmas = []
        for d in range(1, N_DEV):
            peer = (my_pos + d) % N_DEV
            rdma = pltpu.make_async_remote_copy(
                src_ref=ag_send_buf,
                dst_ref=ag_buf.at[d - 1],
                send_sem=ag_send_sems.at[d - 1],
                recv_sem=ag_recv_sems.at[d - 1],
                device_id=(peer,),
                device_id_type=pl.DeviceIdType.MESH,
            )
            rdma.start()
            ag_rdmas.append(rdma)

        out_ref[pl.ds(my_pos * TB, TB), :] = acc

        for d in range(1, N_DEV):
            src = (my_pos - d) % N_DEV
            ag_rdmas[d - 1].wait_recv()
            out_ref[pl.ds(src * TB, TB), :] = (
                ag_buf[d - 1, :, :].astype(jnp.float32))

        for d in range(1, N_DEV):
            rs_rdmas[d - 1].wait_send()
            ag_rdmas[d - 1].wait_send()

    return pl.pallas_call(
        body,
        out_shape=jax.ShapeDtypeStruct((T, D), jnp.float32),
        in_specs=[
            pl.BlockSpec(memory_space=pltpu.VMEM),
            pl.BlockSpec(memory_space=pltpu.VMEM),
        ],
        out_specs=pl.BlockSpec(memory_space=pltpu.VMEM),
        scratch_shapes=[
            pltpu.VMEM((N_DEV - 1, TB, D), jnp.bfloat16),
            pltpu.VMEM((N_DEV - 1, TB, D), jnp.bfloat16),
            pltpu.VMEM((TB, D), jnp.bfloat16),
            pltpu.VMEM((N_DEV - 1, TB, D), jnp.bfloat16),
            pltpu.SemaphoreType.DMA((N_DEV - 1,)),
            pltpu.SemaphoreType.DMA((N_DEV - 1,)),
            pltpu.SemaphoreType.DMA((N_DEV - 1,)),
            pltpu.SemaphoreType.DMA((N_DEV - 1,)),
        ],
        compiler_params=pltpu.CompilerParams(collective_id=0),
    )(ids2, E)
